# speedup vs baseline: 1.4615x; 1.4615x over previous
import functools

import jax
import jax.numpy as jnp
from jax import lax
from jax.experimental import pallas as pl
from jax.experimental.pallas import tpu as pltpu
from jax.experimental.pallas import tpu_sc as plsc

EPS = 1e-12

NC = 2
NS = 16
NW = NC * NS


def _sc_gather(table, ids_flat, chunk):
    n, h = ids_flat.shape[0], table.shape[1]
    rows_per_w = n // NW
    nchunk = rows_per_w // chunk
    mesh = plsc.VectorSubcoreMesh(
        core_axis_name="c", subcore_axis_name="s", num_cores=NC, num_subcores=NS
    )

    @functools.partial(
        pl.kernel,
        mesh=mesh,
        out_type=jax.ShapeDtypeStruct((n, h), jnp.float32),
        scratch_types=[
            pltpu.VMEM((chunk,), jnp.int32),
            pltpu.VMEM((chunk, h), jnp.float32),
            pltpu.SemaphoreType.DMA,
        ],
    )
    def gather_kernel(ids_hbm, table_hbm, out_hbm, idx_v, rows_v, sem):
        wid = lax.axis_index("s") * NC + lax.axis_index("c")
        base = wid * rows_per_w
        for c in range(nchunk):
            rb = base + c * chunk
            pltpu.sync_copy(ids_hbm.at[pl.ds(rb, chunk)], idx_v)
            pltpu.async_copy(table_hbm.at[idx_v], rows_v, sem).wait()
            pltpu.sync_copy(rows_v, out_hbm.at[pl.ds(rb, chunk)])

    return gather_kernel(ids_flat, table)


def _tc_fuse(gathered, ttf, pos_emb, type_emb, gamma2, beta2, br):
    n, h = gathered.shape
    seq = pos_emb.shape[0]
    blocks_per_seq = seq // br

    def body(g_ref, tt_ref, pos_ref, type_ref, gam_ref, bet_ref, o_ref):
        x = g_ref[...] + pos_ref[...]
        ttv = tt_ref[...]
        t0 = type_ref[0:1, :]
        t1 = type_ref[1:2, :]
        x = x + t0 + ttv * (t1 - t0)
        mean = jnp.mean(x, axis=-1, keepdims=True)
        xc = x - mean
        var = jnp.mean(xc * xc, axis=-1, keepdims=True)
        inv = lax.rsqrt(var + EPS)
        o_ref[...] = xc * inv * gam_ref[...] + bet_ref[...]

    return pl.pallas_call(
        body,
        grid=(n // br,),
        in_specs=[
            pl.BlockSpec((br, h), lambda g: (g, 0)),
            pl.BlockSpec((br, 1), lambda g: (g, 0)),
            pl.BlockSpec((br, h), lambda g: (g % blocks_per_seq, 0)),
            pl.BlockSpec((2, h), lambda g: (0, 0)),
            pl.BlockSpec((1, h), lambda g: (0, 0)),
            pl.BlockSpec((1, h), lambda g: (0, 0)),
        ],
        out_specs=pl.BlockSpec((br, h), lambda g: (g, 0)),
        out_shape=jax.ShapeDtypeStruct((n, h), jnp.float32),
    )(gathered, ttf, pos_emb, type_emb, gamma2, beta2)


def kernel(input_ids, token_type_ids, word_emb, pos_emb, type_emb, gamma, beta):
    b, s = input_ids.shape
    h = word_emb.shape[1]
    n = b * s
    ids_flat = input_ids.reshape(n).astype(jnp.int32)
    gathered = _sc_gather(word_emb, ids_flat, chunk=64)
    ttf = token_type_ids.reshape(n, 1).astype(jnp.float32)
    out = _tc_fuse(
        gathered,
        ttf,
        pos_emb,
        type_emb,
        gamma.reshape(1, h),
        beta.reshape(1, h),
        br=256,
    )
    return out.reshape(b, s, h)


# speedup vs baseline: 1.5121x; 1.0347x over previous
import functools

import jax
import jax.numpy as jnp
from jax import lax
from jax.experimental import pallas as pl
from jax.experimental.pallas import tpu as pltpu
from jax.experimental.pallas import tpu_sc as plsc

EPS = 1e-12

NC = 2
NS = 16
NW = NC * NS


def _sc_gather(table, ids_flat, chunk):
    n, h = ids_flat.shape[0], table.shape[1]
    rows_per_w = n // NW
    nchunk = rows_per_w // chunk
    mesh = plsc.VectorSubcoreMesh(
        core_axis_name="c", subcore_axis_name="s", num_cores=NC, num_subcores=NS
    )

    @functools.partial(
        pl.kernel,
        mesh=mesh,
        out_type=jax.ShapeDtypeStruct((n, h), jnp.float32),
        scratch_types=[
            pltpu.VMEM((rows_per_w,), jnp.int32),
            pltpu.VMEM((2, chunk, h), jnp.float32),
            pltpu.SemaphoreType.DMA((2,)),
            pltpu.SemaphoreType.DMA((2,)),
        ],
    )
    def gather_kernel(ids_hbm, table_hbm, out_hbm, idx_v, rows_v, gsem, wsem):
        wid = lax.axis_index("s") * NC + lax.axis_index("c")
        base = wid * rows_per_w
        pltpu.sync_copy(ids_hbm.at[pl.ds(base, rows_per_w)], idx_v)

        def start_gather(c):
            pltpu.async_copy(
                table_hbm.at[idx_v.at[pl.ds(c * chunk, chunk)]],
                rows_v.at[c % 2],
                gsem.at[c % 2],
            )

        start_gather(0)
        for c in range(nchunk):
            b = c % 2
            pltpu.make_async_copy(
                table_hbm.at[idx_v.at[pl.ds(c * chunk, chunk)]],
                rows_v.at[b],
                gsem.at[b],
            ).wait()
            if c + 1 < nchunk:
                if c >= 1:
                    pltpu.make_async_copy(
                        rows_v.at[1 - b],
                        out_hbm.at[pl.ds(base + (c - 1) * chunk, chunk)],
                        wsem.at[1 - b],
                    ).wait()
                start_gather(c + 1)
            pltpu.async_copy(
                rows_v.at[b],
                out_hbm.at[pl.ds(base + c * chunk, chunk)],
                wsem.at[b],
            )
        last = nchunk - 1
        pltpu.make_async_copy(
            rows_v.at[(last - 1) % 2],
            out_hbm.at[pl.ds(base + (last - 1) * chunk, chunk)],
            wsem.at[(last - 1) % 2],
        ).wait()
        pltpu.make_async_copy(
            rows_v.at[last % 2],
            out_hbm.at[pl.ds(base + last * chunk, chunk)],
            wsem.at[last % 2],
        ).wait()

    return gather_kernel(ids_flat, table)


def _tc_fuse(gathered, ttf, pos_emb, type_emb, gamma2, beta2, br):
    n, h = gathered.shape
    seq = pos_emb.shape[0]
    blocks_per_seq = seq // br

    def body(g_ref, tt_ref, pos_ref, type_ref, gam_ref, bet_ref, o_ref):
        x = g_ref[...] + pos_ref[...]
        ttv = tt_ref[...]
        t0 = type_ref[0:1, :]
        t1 = type_ref[1:2, :]
        x = x + t0 + ttv * (t1 - t0)
        mean = jnp.mean(x, axis=-1, keepdims=True)
        xc = x - mean
        var = jnp.mean(xc * xc, axis=-1, keepdims=True)
        inv = lax.rsqrt(var + EPS)
        o_ref[...] = xc * inv * gam_ref[...] + bet_ref[...]

    nb = n // (br * blocks_per_seq)
    return pl.pallas_call(
        body,
        grid=(blocks_per_seq, nb),
        in_specs=[
            pl.BlockSpec((br, h), lambda p, b: (b * blocks_per_seq + p, 0)),
            pl.BlockSpec((br, 1), lambda p, b: (b * blocks_per_seq + p, 0)),
            pl.BlockSpec((br, h), lambda p, b: (p, 0)),
            pl.BlockSpec((2, h), lambda p, b: (0, 0)),
            pl.BlockSpec((1, h), lambda p, b: (0, 0)),
            pl.BlockSpec((1, h), lambda p, b: (0, 0)),
        ],
        out_specs=pl.BlockSpec((br, h), lambda p, b: (b * blocks_per_seq + p, 0)),
        out_shape=jax.ShapeDtypeStruct((n, h), jnp.float32),
    )(gathered, ttf, pos_emb, type_emb, gamma2, beta2)


def kernel(input_ids, token_type_ids, word_emb, pos_emb, type_emb, gamma, beta):
    b, s = input_ids.shape
    h = word_emb.shape[1]
    n = b * s
    ids_flat = input_ids.reshape(n).astype(jnp.int32)
    gathered = _sc_gather(word_emb, ids_flat, chunk=64)
    ttf = token_type_ids.reshape(n, 1).astype(jnp.float32)
    out = _tc_fuse(
        gathered,
        ttf,
        pos_emb,
        type_emb,
        gamma.reshape(1, h),
        beta.reshape(1, h),
        br=256,
    )
    return out.reshape(b, s, h)
